# Initial kernel scaffold; baseline (speedup 1.0000x reference)
import functools

import jax
import jax.numpy as jnp
from jax import lax
from jax.experimental import pallas as pl
from jax.experimental.pallas import tpu as pltpu
from jax.experimental.pallas import tpu_sc as plsc

N = 10000
E = 320000
D = 128
NG = 16

W = 128
NW = 32
E_PAD = 327680
NWIN = E_PAD // W
WIN_PER_TILE = NWIN // NW
NPAD = 10016
DUMMY_ROW = 10008
ROWS_PER_TILE = NPAD // 16

BE = 512
BN = 1000

_mesh = plsc.VectorSubcoreMesh(core_axis_name="c", subcore_axis_name="s")


def _sc_gather_pair(table, idx_a, idx_b, *, interpret=False):
    out_t = jax.ShapeDtypeStruct((E_PAD, D), jnp.float32)

    @functools.partial(
        pl.kernel,
        out_type=(out_t, out_t),
        mesh=_mesh,
        scratch_types=[],
        interpret=interpret,
    )
    def k(table_hbm, ia_hbm, ib_hbm, oa_hbm, ob_hbm):
        def body(i_v, o_v):
            pltpu.sync_copy(table_hbm.at[i_v.at[0]], o_v)

        for i_hbm, o_hbm in ((ia_hbm, oa_hbm), (ib_hbm, ob_hbm)):
            pltpu.emit_pipeline(
                body,
                grid=(NWIN,),
                in_specs=[pl.BlockSpec((1, W), lambda i: (i, 0))],
                out_specs=[pl.BlockSpec((W, D), lambda i: (i, 0))],
                core_axis_name=("c", "s"),
                dimension_semantics=(pltpu.PARALLEL,),
            )(i_hbm, o_hbm)

    return k(table, idx_a, idx_b)


def _sc_scatter_add(payload, idx, zeros, ncols, *, interpret=False):
    @functools.partial(
        pl.kernel,
        out_type=jax.ShapeDtypeStruct((2, NPAD, ncols), jnp.float32),
        mesh=_mesh,
        scratch_types=[pltpu.VMEM_SHARED((NPAD, ncols), jnp.float32)],
        interpret=interpret,
    )
    def k(pay_hbm, idx_hbm, zero_hbm, out_hbm, acc):
        cid = lax.axis_index("c")
        sid = lax.axis_index("s")
        rows = pl.ds(sid * ROWS_PER_TILE, ROWS_PER_TILE)
        pltpu.sync_copy(zero_hbm.at[rows], acc.at[rows])
        plsc.subcore_barrier()

        def body(p_v, i_v):
            pltpu.sync_copy(p_v, acc.at[i_v.at[0]], add=True)

        pltpu.emit_pipeline(
            body,
            grid=(NWIN,),
            in_specs=[
                pl.BlockSpec((W, ncols), lambda i: (i, 0)),
                pl.BlockSpec((1, W), lambda i: (i, 0)),
            ],
            out_specs=[],
            core_axis_name=("c", "s"),
            dimension_semantics=(pltpu.PARALLEL,),
        )(pay_hbm, idx_hbm)

        plsc.subcore_barrier()
        pltpu.sync_copy(acc.at[rows], out_hbm.at[cid].at[rows])

    return k(payload, idx, zeros)


def _edge_kernel(xs_ref, xd_ref, ea_ref, w1x_ref, w1d_ref, w1e_ref, b1_ref,
                 w2_ref, b2_ref, uvec_ref, s1_ref, s2_ref, *, ein):
    xs = xs_ref[...]
    xd = xd_ref[...]
    ea = ea_ref[...][:, :ein]
    dot = functools.partial(jnp.dot, preferred_element_type=jnp.float32)
    t = dot(xs, w1x_ref[...]) + dot(xd, w1d_ref[...]) + dot(ea, w1e_ref[...])
    t = jax.nn.relu(t + b1_ref[...])
    eo = dot(t, w2_ref[...]) + b2_ref[...]
    us = uvec_ref[0:1, :]
    ud = uvec_ref[1:2, :]
    ve = uvec_ref[2:3, :]
    l = ((xs * us).sum(1, keepdims=True) + (xd * ud).sum(1, keepdims=True)
         + (eo * ve).sum(1, keepdims=True))
    l = jnp.where(l > 0, l, 0.2 * l)
    w = jnp.exp(l)
    extra = jnp.concatenate(
        [w, jnp.ones((BE, 1), jnp.float32), jnp.zeros((BE, 14), jnp.float32)],
        axis=1)
    s1_ref[...] = jnp.concatenate([eo, extra], axis=1)
    s2_ref[...] = w * xs


def _tc_edge_pass(xs, xd, ea, p, uvec, *, interpret=False):
    ein = D if ea.shape[1] == 144 else ea.shape[1]
    ea_bcols = ea.shape[1]
    w1x = p['eW1'][:D]
    w1d = p['eW1'][D:2 * D]
    w1e = p['eW1'][2 * D:]
    return pl.pallas_call(
        functools.partial(_edge_kernel, ein=ein),
        grid=(E // BE,),
        in_specs=[
            pl.BlockSpec((BE, D), lambda i: (i, 0)),
            pl.BlockSpec((BE, D), lambda i: (i, 0)),
            pl.BlockSpec((BE, ea_bcols), lambda i: (i, 0)),
            pl.BlockSpec((D, D), lambda i: (0, 0)),
            pl.BlockSpec((D, D), lambda i: (0, 0)),
            pl.BlockSpec((ein, D), lambda i: (0, 0)),
            pl.BlockSpec((1, D), lambda i: (0, 0)),
            pl.BlockSpec((D, D), lambda i: (0, 0)),
            pl.BlockSpec((1, D), lambda i: (0, 0)),
            pl.BlockSpec((3, D), lambda i: (0, 0)),
        ],
        out_specs=[
            pl.BlockSpec((BE, 144), lambda i: (i, 0)),
            pl.BlockSpec((BE, D), lambda i: (i, 0)),
        ],
        out_shape=[
            jax.ShapeDtypeStruct((E_PAD, 144), jnp.float32),
            jax.ShapeDtypeStruct((E_PAD, D), jnp.float32),
        ],
        interpret=interpret,
    )(xs, xd, ea, w1x, w1d, w1e, p['eb1'].reshape(1, D), p['eW2'],
      p['eb2'].reshape(1, D), uvec)


def _node_kernel(x_ref, s1a_ref, s1b_ref, s2a_ref, s2b_ref, gw_ref, uvec_ref,
                 gb_ref, out_ref, stats_ref):
    i = pl.program_id(0)
    x = x_ref[...]
    acc1 = s1a_ref[0] + s1b_ref[0]
    agg2 = s2a_ref[0] + s2b_ref[0]
    loop_sum = acc1[:, :D]
    wsum = acc1[:, D:D + 1]
    deg = acc1[:, D + 1:D + 2]
    loop_attr = loop_sum / jnp.maximum(deg, 1.0)
    dot = functools.partial(jnp.dot, preferred_element_type=jnp.float32)
    h = dot(x, gw_ref[...])
    usd = uvec_ref[0:1, :] + uvec_ref[1:2, :]
    ve = uvec_ref[2:3, :]
    l_self = (x * usd).sum(1, keepdims=True) + (loop_attr * ve).sum(
        1, keepdims=True)
    l_self = jnp.where(l_self > 0, l_self, 0.2 * l_self)
    w_self = jnp.exp(l_self)
    out_num = dot(agg2, gw_ref[...]) + w_self * h
    s = wsum + w_self
    out = out_num / (s + 1e-16) + gb_ref[...]
    out_ref[...] = out
    st = jnp.concatenate(
        [out.sum(0, keepdims=True), (out * out).sum(0, keepdims=True),
         jnp.zeros((6, D), jnp.float32)], axis=0)

    @pl.when(i == 0)
    def _():
        stats_ref[...] = st

    @pl.when(i != 0)
    def _():
        stats_ref[...] += st


def _tc_node_pass(x, s1p, s2p, p, uvec, *, interpret=False):
    return pl.pallas_call(
        _node_kernel,
        grid=(N // BN,),
        in_specs=[
            pl.BlockSpec((BN, D), lambda i: (i, 0)),
            pl.BlockSpec((1, BN, 144), lambda i: (0, i, 0)),
            pl.BlockSpec((1, BN, 144), lambda i: (1, i, 0)),
            pl.BlockSpec((1, BN, D), lambda i: (0, i, 0)),
            pl.BlockSpec((1, BN, D), lambda i: (1, i, 0)),
            pl.BlockSpec((D, D), lambda i: (0, 0)),
            pl.BlockSpec((3, D), lambda i: (0, 0)),
            pl.BlockSpec((1, D), lambda i: (0, 0)),
        ],
        out_specs=[
            pl.BlockSpec((BN, D), lambda i: (i, 0)),
            pl.BlockSpec((8, D), lambda i: (0, 0)),
        ],
        out_shape=[
            jax.ShapeDtypeStruct((N, D), jnp.float32),
            jax.ShapeDtypeStruct((8, D), jnp.float32),
        ],
        interpret=interpret,
    )(x, s1p, s1p, s2p, s2p, p['gW'], uvec, p['gb'].reshape(1, D))


def _bn_kernel(out_ref, stats_ref, g_ref, b_ref, xn_ref):
    mu = stats_ref[0:1, :] / N
    var = stats_ref[1:2, :] / N - mu * mu
    xn = (out_ref[...] - mu) / jnp.sqrt(var + 1e-5) * g_ref[...] + b_ref[...]
    xn_ref[...] = jax.nn.relu(xn)


def _tc_bn_pass(out, stats, p, *, interpret=False):
    return pl.pallas_call(
        _bn_kernel,
        grid=(N // BN,),
        in_specs=[
            pl.BlockSpec((BN, D), lambda i: (i, 0)),
            pl.BlockSpec((8, D), lambda i: (0, 0)),
            pl.BlockSpec((1, D), lambda i: (0, 0)),
            pl.BlockSpec((1, D), lambda i: (0, 0)),
        ],
        out_specs=pl.BlockSpec((BN, D), lambda i: (i, 0)),
        out_shape=jax.ShapeDtypeStruct((N, D), jnp.float32),
        interpret=interpret,
    )(out, stats, p['bn_g'].reshape(1, D), p['bn_b'].reshape(1, D))


def _pool_kernel(x_ref, b_ref, gsum_ref, cnt_ref):
    i = pl.program_id(0)
    b = b_ref[0]
    oh = (lax.broadcasted_iota(jnp.int32, (NG, BN), 0) == b).astype(
        jnp.float32)
    gs = jnp.dot(oh, x_ref[...], preferred_element_type=jnp.float32)
    ct = oh.sum(1, keepdims=True)

    @pl.when(i == 0)
    def _():
        gsum_ref[...] = gs
        cnt_ref[...] = ct

    @pl.when(i != 0)
    def _():
        gsum_ref[...] += gs
        cnt_ref[...] += ct


def _tc_pool(x, batch3d, *, interpret=False):
    return pl.pallas_call(
        _pool_kernel,
        grid=(N // BN,),
        in_specs=[
            pl.BlockSpec((BN, D), lambda i: (i, 0)),
            pl.BlockSpec((1, 1, BN), lambda i: (i, 0, 0)),
        ],
        out_specs=[
            pl.BlockSpec((NG, D), lambda i: (0, 0)),
            pl.BlockSpec((NG, 1), lambda i: (0, 0)),
        ],
        out_shape=[
            jax.ShapeDtypeStruct((NG, D), jnp.float32),
            jax.ShapeDtypeStruct((NG, 1), jnp.float32),
        ],
        interpret=interpret,
    )(x, batch3d)


def _mlp_kernel(gsum_ref, cnt_ref, w1_ref, b1_ref, w2_ref, b2_ref, w3_ref,
                b3_ref, o_ref):
    g = gsum_ref[...] / jnp.maximum(cnt_ref[...], 1.0)
    dot = functools.partial(jnp.dot, preferred_element_type=jnp.float32)
    g = jax.nn.relu(dot(g, w1_ref[...]) + b1_ref[...])
    g = jax.nn.relu(dot(g, w2_ref[...]) + b2_ref[...])
    o_ref[...] = dot(g, w3_ref[...]) + b3_ref[...]


def _tc_mlp(gsum, cnt, m, *, interpret=False):
    return pl.pallas_call(
        _mlp_kernel,
        out_shape=jax.ShapeDtypeStruct((NG, 1), jnp.float32),
        interpret=interpret,
    )(gsum, cnt, m['W1'], m['b1'].reshape(1, D), m['W2'],
      m['b2'].reshape(1, D // 2), m['W3'], m['b3'].reshape(1, 1))


def kernel(x, edge_index, edge_attr, batch, params):
    src = edge_index[0]
    dst = edge_index[1]
    padg = jnp.zeros((E_PAD - E,), jnp.int32)
    pads = jnp.full((E_PAD - E,), DUMMY_ROW, jnp.int32)
    src_g = jnp.concatenate([src, padg]).reshape(NWIN, W)
    dst_g = jnp.concatenate([dst, padg]).reshape(NWIN, W)
    dst_s = jnp.concatenate([dst, pads]).reshape(NWIN, W)
    batch3d = batch.reshape(N // BN, 1, BN)
    zeros144 = jnp.zeros((NPAD, 144), jnp.float32)
    zeros128 = jnp.zeros((NPAD, D), jnp.float32)

    ea = edge_attr
    for i in range(3):
        p = params['l%d' % i]
        uvec = jnp.stack([p['gW'] @ p['att_src'], p['gW'] @ p['att_dst'],
                          p['gWe'] @ p['att_edge']])
        xs, xd = _sc_gather_pair(x, src_g, dst_g)
        s1, s2 = _tc_edge_pass(xs, xd, ea, p, uvec)
        s1p = _sc_scatter_add(s1, dst_s, zeros144, 144)
        s2p = _sc_scatter_add(s2, dst_s, zeros128, D)
        out, stats = _tc_node_pass(x, s1p, s2p, p, uvec)
        x = _tc_bn_pass(out, stats, p)
        ea = s1
    gsum, cnt = _tc_pool(x, batch3d)
    return _tc_mlp(gsum, cnt, params['mlp'])


# speedup vs baseline: 9.0729x; 9.0729x over previous
import functools

import jax
import jax.numpy as jnp
from jax import lax
from jax.experimental import pallas as pl
from jax.experimental.pallas import tpu as pltpu
from jax.experimental.pallas import tpu_sc as plsc

N = 10000
E = 320000
D = 128
NG = 16

W = 128
NW = 32
E_PAD = 327680
NWIN = E_PAD // W
WIN_PER_TILE = NWIN // NW
NPAD = 10112
DUMMY_ROW = 10008
ROWS_PER_TILE = NPAD // 16

BE = 512
BN = 1000

@functools.cache
def _mesh():
    return plsc.VectorSubcoreMesh(core_axis_name="c", subcore_axis_name="s",
                                  num_cores=2, num_subcores=16)


def _sc_gather_pair(table, idx_a, idx_b, *, interpret=False):
    out_t = jax.ShapeDtypeStruct((E_PAD, D), jnp.float32)

    @functools.partial(
        pl.kernel,
        out_type=(out_t, out_t),
        mesh=_mesh(),
        scratch_types=[],
        interpret=interpret,
    )
    def k(table_hbm, ia_hbm, ib_hbm, oa_hbm, ob_hbm):
        def body(i_v, o_v):
            pltpu.sync_copy(table_hbm.at[i_v.at[0]], o_v)

        for i_hbm, o_hbm in ((ia_hbm, oa_hbm), (ib_hbm, ob_hbm)):
            pltpu.emit_pipeline(
                body,
                grid=(NWIN,),
                in_specs=[pl.BlockSpec((1, W), lambda i: (i, 0))],
                out_specs=[pl.BlockSpec((W, D), lambda i: (i, 0))],
                core_axis_name=("c", "s"),
                dimension_semantics=(pltpu.PARALLEL,),
            )(i_hbm, o_hbm)

    return k(table, idx_a, idx_b)


def _sc_scatter_add(payload, idx, zeros, ncols, *, interpret=False):
    @functools.partial(
        pl.kernel,
        out_type=jax.ShapeDtypeStruct((2, NPAD, ncols), jnp.float32),
        mesh=_mesh(),
        scratch_types=[pltpu.VMEM_SHARED((NPAD, ncols), jnp.float32)],
        interpret=interpret,
    )
    def k(pay_hbm, idx_hbm, zero_hbm, out_hbm, acc):
        cid = lax.axis_index("c")
        sid = lax.axis_index("s")
        rows = pl.ds(sid * ROWS_PER_TILE, ROWS_PER_TILE)
        pltpu.sync_copy(zero_hbm.at[rows], acc.at[rows])
        plsc.subcore_barrier()

        def body(p_v, i_v):
            pltpu.sync_copy(p_v, acc.at[i_v.at[0]], add=True)

        pltpu.emit_pipeline(
            body,
            grid=(NWIN,),
            in_specs=[
                pl.BlockSpec((W, ncols), lambda i: (i, 0)),
                pl.BlockSpec((1, W), lambda i: (i, 0)),
            ],
            out_specs=[],
            core_axis_name=("c", "s"),
            dimension_semantics=(pltpu.PARALLEL,),
        )(pay_hbm, idx_hbm)

        plsc.subcore_barrier()
        pltpu.sync_copy(acc.at[rows], out_hbm.at[cid].at[rows])

    return k(payload, idx, zeros)


def _edge_kernel(xs_ref, xd_ref, ea_ref, w1x_ref, w1d_ref, w1e_ref, b1_ref,
                 w2_ref, b2_ref, uvec_ref, eo_ref, scal_ref, s2_ref, *, ein):
    xs = xs_ref[...]
    xd = xd_ref[...]
    ea = ea_ref[...][:, :ein]
    dot = functools.partial(jnp.dot, preferred_element_type=jnp.float32,
                            precision=lax.Precision.HIGHEST)
    t = dot(xs, w1x_ref[...]) + dot(xd, w1d_ref[...]) + dot(ea, w1e_ref[...])
    t = jax.nn.relu(t + b1_ref[...])
    eo = dot(t, w2_ref[...]) + b2_ref[...]
    us = uvec_ref[0:1, :]
    ud = uvec_ref[1:2, :]
    ve = uvec_ref[2:3, :]
    q = (eo * ve).sum(1, keepdims=True)
    l = ((xs * us).sum(1, keepdims=True) + (xd * ud).sum(1, keepdims=True)
         + q)
    l = jnp.where(l > 0, l, 0.2 * l)
    w = jnp.exp(l)
    eo_ref[...] = eo
    scal_ref[...] = jnp.concatenate(
        [q, w, jnp.ones((BE, 1), jnp.float32),
         jnp.zeros((BE, 125), jnp.float32)], axis=1)
    s2_ref[...] = w * xs


def _tc_edge_pass(xs, xd, ea, p, uvec, *, interpret=False):
    ein = ea.shape[1]
    ea_bcols = ea.shape[1]
    w1x = p['eW1'][:D]
    w1d = p['eW1'][D:2 * D]
    w1e = p['eW1'][2 * D:]
    return pl.pallas_call(
        functools.partial(_edge_kernel, ein=ein),
        grid=(E // BE,),
        in_specs=[
            pl.BlockSpec((BE, D), lambda i: (i, 0)),
            pl.BlockSpec((BE, D), lambda i: (i, 0)),
            pl.BlockSpec((BE, ea_bcols), lambda i: (i, 0)),
            pl.BlockSpec((D, D), lambda i: (0, 0)),
            pl.BlockSpec((D, D), lambda i: (0, 0)),
            pl.BlockSpec((ein, D), lambda i: (0, 0)),
            pl.BlockSpec((1, D), lambda i: (0, 0)),
            pl.BlockSpec((D, D), lambda i: (0, 0)),
            pl.BlockSpec((1, D), lambda i: (0, 0)),
            pl.BlockSpec((3, D), lambda i: (0, 0)),
        ],
        out_specs=[
            pl.BlockSpec((BE, D), lambda i: (i, 0)),
            pl.BlockSpec((BE, D), lambda i: (i, 0)),
            pl.BlockSpec((BE, D), lambda i: (i, 0)),
        ],
        out_shape=[
            jax.ShapeDtypeStruct((E_PAD, D), jnp.float32),
            jax.ShapeDtypeStruct((E_PAD, D), jnp.float32),
            jax.ShapeDtypeStruct((E_PAD, D), jnp.float32),
        ],
        interpret=interpret,
    )(xs, xd, ea, w1x, w1d, w1e, p['eb1'].reshape(1, D), p['eW2'],
      p['eb2'].reshape(1, D), uvec)


def _node_kernel(x_ref, s1a_ref, s1b_ref, s2a_ref, s2b_ref, gw_ref, uvec_ref,
                 gb_ref, out_ref, stats_ref):
    i = pl.program_id(0)
    x = x_ref[...]
    acc1 = s1a_ref[0] + s1b_ref[0]
    agg2 = s2a_ref[0] + s2b_ref[0]
    q_sum = acc1[:, 0:1]
    wsum = acc1[:, 1:2]
    deg = acc1[:, 2:3]
    dot = functools.partial(jnp.dot, preferred_element_type=jnp.float32,
                            precision=lax.Precision.HIGHEST)
    h = dot(x, gw_ref[...])
    usd = uvec_ref[0:1, :] + uvec_ref[1:2, :]
    l_self = (x * usd).sum(1, keepdims=True) + q_sum / jnp.maximum(deg, 1.0)
    l_self = jnp.where(l_self > 0, l_self, 0.2 * l_self)
    w_self = jnp.exp(l_self)
    out_num = dot(agg2, gw_ref[...]) + w_self * h
    s = wsum + w_self
    out = out_num / (s + 1e-16) + gb_ref[...]
    out_ref[...] = out
    st = jnp.concatenate(
        [out.sum(0, keepdims=True), (out * out).sum(0, keepdims=True),
         jnp.zeros((6, D), jnp.float32)], axis=0)

    @pl.when(i == 0)
    def _():
        stats_ref[...] = st

    @pl.when(i != 0)
    def _():
        stats_ref[...] += st


def _tc_node_pass(x, s1p, s2p, p, uvec, *, interpret=False):
    return pl.pallas_call(
        _node_kernel,
        grid=(N // BN,),
        in_specs=[
            pl.BlockSpec((BN, D), lambda i: (i, 0)),
            pl.BlockSpec((1, BN, D), lambda i: (0, i, 0)),
            pl.BlockSpec((1, BN, D), lambda i: (1, i, 0)),
            pl.BlockSpec((1, BN, D), lambda i: (0, i, 0)),
            pl.BlockSpec((1, BN, D), lambda i: (1, i, 0)),
            pl.BlockSpec((D, D), lambda i: (0, 0)),
            pl.BlockSpec((3, D), lambda i: (0, 0)),
            pl.BlockSpec((1, D), lambda i: (0, 0)),
        ],
        out_specs=[
            pl.BlockSpec((BN, D), lambda i: (i, 0)),
            pl.BlockSpec((8, D), lambda i: (0, 0)),
        ],
        out_shape=[
            jax.ShapeDtypeStruct((N, D), jnp.float32),
            jax.ShapeDtypeStruct((8, D), jnp.float32),
        ],
        interpret=interpret,
    )(x, s1p, s1p, s2p, s2p, p['gW'], uvec, p['gb'].reshape(1, D))


def _bn_kernel(out_ref, stats_ref, g_ref, b_ref, xn_ref):
    mu = stats_ref[0:1, :] / N
    var = stats_ref[1:2, :] / N - mu * mu
    xn = (out_ref[...] - mu) / jnp.sqrt(var + 1e-5) * g_ref[...] + b_ref[...]
    xn_ref[...] = jax.nn.relu(xn)


def _tc_bn_pass(out, stats, p, *, interpret=False):
    return pl.pallas_call(
        _bn_kernel,
        grid=(N // BN,),
        in_specs=[
            pl.BlockSpec((BN, D), lambda i: (i, 0)),
            pl.BlockSpec((8, D), lambda i: (0, 0)),
            pl.BlockSpec((1, D), lambda i: (0, 0)),
            pl.BlockSpec((1, D), lambda i: (0, 0)),
        ],
        out_specs=pl.BlockSpec((BN, D), lambda i: (i, 0)),
        out_shape=jax.ShapeDtypeStruct((N, D), jnp.float32),
        interpret=interpret,
    )(out, stats, p['bn_g'].reshape(1, D), p['bn_b'].reshape(1, D))


def _pool_kernel(x_ref, b_ref, gsum_ref, cnt_ref):
    i = pl.program_id(0)
    b = b_ref[0]
    oh = (lax.broadcasted_iota(jnp.int32, (NG, BN), 0) == b).astype(
        jnp.float32)
    gs = jnp.dot(oh, x_ref[...], preferred_element_type=jnp.float32,
                 precision=lax.Precision.HIGHEST)
    ct = oh.sum(1, keepdims=True)

    @pl.when(i == 0)
    def _():
        gsum_ref[...] = gs
        cnt_ref[...] = ct

    @pl.when(i != 0)
    def _():
        gsum_ref[...] += gs
        cnt_ref[...] += ct


def _tc_pool(x, batch3d, *, interpret=False):
    return pl.pallas_call(
        _pool_kernel,
        grid=(N // BN,),
        in_specs=[
            pl.BlockSpec((BN, D), lambda i: (i, 0)),
            pl.BlockSpec((1, 1, BN), lambda i: (i, 0, 0)),
        ],
        out_specs=[
            pl.BlockSpec((NG, D), lambda i: (0, 0)),
            pl.BlockSpec((NG, 1), lambda i: (0, 0)),
        ],
        out_shape=[
            jax.ShapeDtypeStruct((NG, D), jnp.float32),
            jax.ShapeDtypeStruct((NG, 1), jnp.float32),
        ],
        interpret=interpret,
    )(x, batch3d)


def _mlp_kernel(gsum_ref, cnt_ref, w1_ref, b1_ref, w2_ref, b2_ref, w3_ref,
                b3_ref, o_ref):
    g = gsum_ref[...] / jnp.maximum(cnt_ref[...], 1.0)
    dot = functools.partial(jnp.dot, preferred_element_type=jnp.float32,
                            precision=lax.Precision.HIGHEST)
    g = jax.nn.relu(dot(g, w1_ref[...]) + b1_ref[...])
    g = jax.nn.relu(dot(g, w2_ref[...]) + b2_ref[...])
    o_ref[...] = dot(g, w3_ref[...]) + b3_ref[...]


def _tc_mlp(gsum, cnt, m, *, interpret=False):
    return pl.pallas_call(
        _mlp_kernel,
        out_shape=jax.ShapeDtypeStruct((NG, 1), jnp.float32),
        interpret=interpret,
    )(gsum, cnt, m['W1'], m['b1'].reshape(1, D), m['W2'],
      m['b2'].reshape(1, D // 2), m['W3'], m['b3'].reshape(1, 1))


def kernel(x, edge_index, edge_attr, batch, params):
    src = edge_index[0]
    dst = edge_index[1]
    padg = jnp.zeros((E_PAD - E,), jnp.int32)
    pads = jnp.full((E_PAD - E,), DUMMY_ROW, jnp.int32)
    src_g = jnp.concatenate([src, padg]).reshape(NWIN, W)
    dst_g = jnp.concatenate([dst, padg]).reshape(NWIN, W)
    dst_s = jnp.concatenate([dst, pads]).reshape(NWIN, W)
    batch3d = batch.reshape(N // BN, 1, BN)
    zeros128 = jnp.zeros((NPAD, D), jnp.float32)

    ea = edge_attr
    for i in range(3):
        p = params['l%d' % i]
        uvec = jnp.stack([p['gW'] @ p['att_src'], p['gW'] @ p['att_dst'],
                          p['gWe'] @ p['att_edge']])
        xs, xd = _sc_gather_pair(x, src_g, dst_g)
        eo, scal, s2 = _tc_edge_pass(xs, xd, ea, p, uvec)
        s1p = _sc_scatter_add(scal, dst_s, zeros128, D)
        s2p = _sc_scatter_add(s2, dst_s, zeros128, D)
        out, stats = _tc_node_pass(x, s1p, s2p, p, uvec)
        x = _tc_bn_pass(out, stats, p)
        ea = eo
    gsum, cnt = _tc_pool(x, batch3d)
    return _tc_mlp(gsum, cnt, params['mlp'])
